# baseline (speedup 1.0000x reference)
import functools

import jax
import jax.numpy as jnp
from jax import lax
from jax.experimental import pallas as pl
from jax.experimental.pallas import tpu as pltpu
from jax.experimental.pallas import tpu_sc as plsc

_B = 4096 * 50
_D = 32
_NC = 2
_NS = 16
_NW = _NC * _NS
_BPW = _B // _NW
_CHUNK = 1600
_NCHUNK = _BPW // _CHUNK


def _make_emb_kernel():
  mesh = plsc.VectorSubcoreMesh(core_axis_name="c", subcore_axis_name="s")

  @functools.partial(
      pl.kernel,
      mesh=mesh,
      out_type=jax.ShapeDtypeStruct((_B, _D), jnp.float32),
      compiler_params=pltpu.CompilerParams(use_tc_tiling_on_sc=False),
      scratch_types=[
          pltpu.VMEM((_CHUNK,), jnp.int32),
          pltpu.VMEM((_CHUNK, _D), jnp.float32),
          pltpu.SemaphoreType.DMA,
      ],
  )
  def emb(idx_hbm, table_hbm, out_hbm, idx_v, rows_v, sem):
    wid = lax.axis_index("s") * _NC + lax.axis_index("c")
    base = wid * _BPW
    for ci in range(_NCHUNK):
      off = base + ci * _CHUNK
      pltpu.sync_copy(idx_hbm.at[pl.ds(off, _CHUNK)], idx_v)
      pltpu.async_copy(table_hbm.at[idx_v], rows_v, sem).wait()
      pltpu.sync_copy(rows_v, out_hbm.at[pl.ds(off, _CHUNK)])

  return emb


_emb = _make_emb_kernel()


@jax.jit
def kernel(x, table):
  idx = x.reshape(-1).astype(jnp.int32)
  out = _emb(idx, table)
  return out.reshape(x.shape + (_D,))


# speedup vs baseline: 1.0049x; 1.0049x over previous
import functools

import jax
import jax.numpy as jnp
from jax import lax
from jax.experimental import pallas as pl
from jax.experimental.pallas import tpu as pltpu
from jax.experimental.pallas import tpu_sc as plsc

_B = 4096 * 50
_D = 32
_NC = 2
_NS = 16
_NW = _NC * _NS
_BPW = _B // _NW
_NBUF = 4
_CHUNK = 800
_NCHUNK = _BPW // _CHUNK


def _make_emb_kernel():
  mesh = plsc.VectorSubcoreMesh(core_axis_name="c", subcore_axis_name="s")

  @functools.partial(
      pl.kernel,
      mesh=mesh,
      out_type=jax.ShapeDtypeStruct((_B, _D), jnp.float32),
      compiler_params=pltpu.CompilerParams(use_tc_tiling_on_sc=False),
      scratch_types=(
          [pltpu.VMEM((_BPW,), jnp.int32)]
          + [pltpu.VMEM((_CHUNK, _D), jnp.float32) for _ in range(_NBUF)]
          + [pltpu.SemaphoreType.DMA for _ in range(2 * _NBUF)]
      ),
  )
  def emb(idx_hbm, table_hbm, out_hbm, idx_v, *bufs_and_sems):
    rows = bufs_and_sems[:_NBUF]
    gsem = bufs_and_sems[_NBUF:2 * _NBUF]
    wsem = bufs_and_sems[2 * _NBUF:]
    wid = lax.axis_index("s") * _NC + lax.axis_index("c")
    base = wid * _BPW
    pltpu.sync_copy(idx_hbm.at[pl.ds(base, _BPW)], idx_v)

    gathers = [None] * _NCHUNK
    writes = [None] * _NCHUNK

    def start_gather(ci):
      b = ci % _NBUF
      gathers[ci] = pltpu.async_copy(
          table_hbm.at[idx_v.at[pl.ds(ci * _CHUNK, _CHUNK)]], rows[b], gsem[b])

    for ci in range(_NBUF):
      start_gather(ci)
    for ci in range(_NCHUNK):
      b = ci % _NBUF
      gathers[ci].wait()
      writes[ci] = pltpu.async_copy(
          rows[b], out_hbm.at[pl.ds(base + ci * _CHUNK, _CHUNK)], wsem[b])
      nxt = ci + _NBUF
      if nxt < _NCHUNK:
        writes[ci].wait()
        start_gather(nxt)
    for ci in range(_NCHUNK - _NBUF, _NCHUNK):
      writes[ci].wait()

  return emb


_emb = _make_emb_kernel()


@jax.jit
def kernel(x, table):
  idx = x.reshape(-1).astype(jnp.int32)
  out = _emb(idx, table)
  return out.reshape(x.shape + (_D,))


# speedup vs baseline: 1.1608x; 1.1551x over previous
import functools

import jax
import jax.numpy as jnp
from jax import lax
from jax.experimental import pallas as pl
from jax.experimental.pallas import tpu as pltpu
from jax.experimental.pallas import tpu_sc as plsc

_BATCH = 4096
_SEQ = 50
_D = 32
_NC = 2
_NS = 16
_NW = _NC * _NS
_BT = _BATCH // _NW


def _make_emb_kernel():
  mesh = plsc.VectorSubcoreMesh(core_axis_name="c", subcore_axis_name="s")

  @functools.partial(
      pl.kernel,
      mesh=mesh,
      out_type=jax.ShapeDtypeStruct((_SEQ, _D // 8, _NW, 8, _BT), jnp.float32),
      compiler_params=pltpu.CompilerParams(
          use_tc_tiling_on_sc=False, needs_layout_passes=False),
      scratch_types=[
          pltpu.VMEM((_SEQ, _BT), jnp.int32),
          pltpu.VMEM((_BT, _D), jnp.float32),
          pltpu.VMEM((_BT, _D), jnp.float32),
          pltpu.VMEM((_D, _BT), jnp.float32),
          pltpu.SemaphoreType.DMA,
          pltpu.SemaphoreType.DMA,
      ],
  )
  def emb(idx_hbm, table_hbm, out_hbm, idx_v, rows_a, rows_b, tbuf, sem_a,
          sem_b):
    w = lax.axis_index("s") * _NC + lax.axis_index("c")
    pltpu.sync_copy(idx_hbm.at[:, pl.ds(w * _BT, _BT)], idx_v)
    lane = lax.iota(jnp.int32, 16)

    def start_gather(s, rows, sem):
      pltpu.async_copy(table_hbm.at[idx_v.at[s]], rows, sem)

    def wait_gather(s, rows, sem):
      pltpu.make_async_copy(table_hbm.at[idx_v.at[s]], rows, sem).wait()

    def process(s, rows):
      def per_feature(f, _):
        col = jnp.full((16,), f, jnp.int32)
        for bg in range(_BT // 16):
          vec = plsc.load_gather(rows, [bg * 16 + lane, col])
          tbuf[f, pl.ds(bg * 16, 16)] = vec
        return 0
      lax.fori_loop(0, _D, per_feature, 0)
      for ft in range(_D // 8):
        pltpu.sync_copy(tbuf.at[pl.ds(ft * 8, 8)], out_hbm.at[s, ft, w])

    start_gather(0, rows_a, sem_a)

    def step(i, _):
      s0 = 2 * i
      start_gather(s0 + 1, rows_b, sem_b)
      wait_gather(s0, rows_a, sem_a)
      process(s0, rows_a)

      @pl.when(s0 + 2 < _SEQ)
      def _():
        start_gather(s0 + 2, rows_a, sem_a)

      wait_gather(s0 + 1, rows_b, sem_b)
      process(s0 + 1, rows_b)
      return 0

    lax.fori_loop(0, _SEQ // 2, step, 0)

  return emb


_emb = _make_emb_kernel()


@jax.jit
def kernel(x, table):
  idx = jnp.transpose(x.astype(jnp.int32))
  out5d = _emb(idx, table)
  out = jnp.transpose(out5d, (2, 4, 0, 1, 3))
  return out.reshape(_BATCH, _SEQ, _D)
